# Initial kernel scaffold; baseline (speedup 1.0000x reference)
import functools

import jax
import jax.numpy as jnp
from jax import lax
from jax.experimental import pallas as pl
from jax.experimental.pallas import tpu as pltpu
from jax.experimental.pallas import tpu_sc as plsc

VOCAB_SIZE = 170000
EMB_D = 16
BATCH_N = 16384
SEQ_N = 200

TC_ROWS = VOCAB_SIZE * EMB_D // 128

NUM_CORES = 2
NUM_SUBCORES = 16
NUM_WORKERS = NUM_CORES * NUM_SUBCORES
ROWS_PER_WORKER = BATCH_N // NUM_WORKERS
CHUNK_ROWS = 64
NUM_CHUNKS = ROWS_PER_WORKER // CHUNK_ROWS
CHUNK_ELEMS = CHUNK_ROWS * SEQ_N
NUM_PAIRS = VOCAB_SIZE // 2
LANES = 16
GROUPS_PER_CHUNK = CHUNK_ROWS // LANES
S_UNROLL = 8


def _tc_tw_body(tbl_ref, wmat_ref, b_ref, out_ref):
    out_ref[...] = (
        jnp.dot(tbl_ref[...], wmat_ref[...], preferred_element_type=jnp.float32)
        + b_ref[0, 0]
    )


def _compute_tw(table, W, b):
    table2 = table.reshape(TC_ROWS, 128)
    wmat = jnp.kron(jnp.eye(8, dtype=jnp.float32), W.reshape(EMB_D, 1))
    tw8 = pl.pallas_call(
        _tc_tw_body,
        out_shape=jax.ShapeDtypeStruct((TC_ROWS, 8), jnp.float32),
        in_specs=[
            pl.BlockSpec(memory_space=pltpu.VMEM),
            pl.BlockSpec(memory_space=pltpu.VMEM),
            pl.BlockSpec(memory_space=pltpu.SMEM),
        ],
        out_specs=pl.BlockSpec(memory_space=pltpu.VMEM),
    )(table2, wmat, b.reshape(1, 1).astype(jnp.float32))
    return tw8.reshape(VOCAB_SIZE)


def _sc_body(tw_hbm, x_hbm, out_hbm, twbuf, xb0, xb1, outbuf, sem_tw, sem_x0, sem_x1):
    wid = lax.axis_index("s") * NUM_CORES + lax.axis_index("c")
    row_base = wid * ROWS_PER_WORKER
    elem_base = row_base * SEQ_N

    cp_tw = pltpu.async_copy(tw_hbm, twbuf, sem_tw)
    xbufs = (xb0, xb1)
    sems = (sem_x0, sem_x1)
    copies = [None, None]
    copies[0] = pltpu.async_copy(
        x_hbm.at[pl.ds(elem_base, CHUNK_ELEMS)], xbufs[0], sems[0]
    )
    cp_tw.wait()

    lane = lax.iota(jnp.int32, LANES)
    hi_mask = jnp.int32(-65536)

    for c in range(NUM_CHUNKS):
        cur = c % 2
        nxt = (c + 1) % 2
        if c + 1 < NUM_CHUNKS:
            copies[nxt] = pltpu.async_copy(
                x_hbm.at[pl.ds(elem_base + (c + 1) * CHUNK_ELEMS, CHUNK_ELEMS)],
                xbufs[nxt],
                sems[nxt],
            )
        copies[cur].wait()
        xb = xbufs[cur]
        for g in range(GROUPS_PER_CHUNK):
            base_idx = (g * LANES + lane) * SEQ_N

            def s_step(i, acc, base_idx=base_idx, xb=xb):
                bi = base_idx + i * S_UNROLL
                for k in range(S_UNROLL):
                    tok = plsc.load_gather(xb, [bi + k])
                    pk = plsc.load_gather(twbuf, [lax.shift_right_logical(tok, 1)])
                    lo = lax.shift_left(pk, 16)
                    hi = lax.bitwise_and(pk, hi_mask)
                    bits = jnp.where(lax.bitwise_and(tok, 1) == 0, lo, hi)
                    acc = acc + plsc.bitcast(bits, jnp.float32)
                return acc

            acc = lax.fori_loop(
                0, SEQ_N // S_UNROLL, s_step, jnp.zeros((LANES,), jnp.float32)
            )
            z = acc * jnp.float32(1.0 / SEQ_N)
            res = 1.0 / (1.0 + jnp.exp(-z))
            outbuf[pl.ds(c * CHUNK_ROWS + g * LANES, LANES)] = res

    pltpu.sync_copy(outbuf, out_hbm.at[pl.ds(row_base, ROWS_PER_WORKER)])


@jax.jit
def kernel(x, table, W, b):
    tw = _compute_tw(table, W, b)
    packed = lax.bitcast_convert_type(
        tw.astype(jnp.bfloat16).reshape(NUM_PAIRS, 2), jnp.int32
    )
    x1d = x.reshape(BATCH_N * SEQ_N).astype(jnp.int32)

    mesh = plsc.VectorSubcoreMesh(
        core_axis_name="c",
        subcore_axis_name="s",
        num_cores=NUM_CORES,
        num_subcores=NUM_SUBCORES,
    )
    out1d = pl.kernel(
        _sc_body,
        out_type=jax.ShapeDtypeStruct((BATCH_N,), jnp.float32),
        mesh=mesh,
        scratch_types=[
            pltpu.VMEM((NUM_PAIRS,), jnp.int32),
            pltpu.VMEM((CHUNK_ELEMS,), jnp.int32),
            pltpu.VMEM((CHUNK_ELEMS,), jnp.int32),
            pltpu.VMEM((ROWS_PER_WORKER,), jnp.float32),
            pltpu.SemaphoreType.DMA,
            pltpu.SemaphoreType.DMA,
            pltpu.SemaphoreType.DMA,
        ],
    )(packed, x1d)
    return out1d.reshape(BATCH_N, 1)


# speedup vs baseline: 229.4297x; 229.4297x over previous
import functools

import jax
import jax.numpy as jnp
from jax import lax
from jax.experimental import pallas as pl
from jax.experimental.pallas import tpu as pltpu
from jax.experimental.pallas import tpu_sc as plsc

VOCAB_SIZE = 170000
EMB_D = 16
BATCH_N = 16384
SEQ_N = 200

TC_ROWS = VOCAB_SIZE * EMB_D // 128

NUM_CORES = 2
NUM_SUBCORES = 16
NUM_WORKERS = NUM_CORES * NUM_SUBCORES
ROWS_PER_WORKER = BATCH_N // NUM_WORKERS
CHUNK_ROWS = 64
NUM_CHUNKS = ROWS_PER_WORKER // CHUNK_ROWS
CHUNK_ELEMS = CHUNK_ROWS * SEQ_N
NUM_PAIRS = VOCAB_SIZE // 2
LANES = 16
GROUPS_PER_CHUNK = CHUNK_ROWS // LANES
S_UNROLL = 8


def _tc_tw_body(tbl_ref, wmat_ref, b_ref, out_ref):
    out_ref[...] = (
        jnp.dot(tbl_ref[...], wmat_ref[...], preferred_element_type=jnp.float32)
        + b_ref[0, 0]
    )


def _compute_tw(table, W, b):
    table2 = table.reshape(TC_ROWS, 128)
    wmat = jnp.kron(jnp.eye(8, dtype=jnp.float32), W.reshape(EMB_D, 1))
    tw8 = pl.pallas_call(
        _tc_tw_body,
        out_shape=jax.ShapeDtypeStruct((TC_ROWS, 8), jnp.float32),
        in_specs=[
            pl.BlockSpec(memory_space=pltpu.VMEM),
            pl.BlockSpec(memory_space=pltpu.VMEM),
            pl.BlockSpec(memory_space=pltpu.SMEM),
        ],
        out_specs=pl.BlockSpec(memory_space=pltpu.VMEM),
    )(table2, wmat, b.reshape(1, 1).astype(jnp.float32))
    return tw8.reshape(VOCAB_SIZE)


def _sc_body(tw_hbm, x_hbm, out_hbm, twbuf, xb0, xb1, outbuf, sem_tw, sem_x0, sem_x1):
    wid = lax.axis_index("s") * NUM_CORES + lax.axis_index("c")
    row_base = wid * ROWS_PER_WORKER
    elem_base = row_base * SEQ_N

    cp_tw = pltpu.async_copy(tw_hbm, twbuf, sem_tw)
    xbufs = (xb0, xb1)
    sems = (sem_x0, sem_x1)
    copies = [None, None]
    copies[0] = pltpu.async_copy(
        x_hbm.at[pl.ds(elem_base, CHUNK_ELEMS)], xbufs[0], sems[0]
    )
    cp_tw.wait()

    lane = lax.iota(jnp.int32, LANES)
    hi_mask = jnp.int32(-65536)

    for c in range(NUM_CHUNKS):
        cur = c % 2
        nxt = (c + 1) % 2
        if c + 1 < NUM_CHUNKS:
            copies[nxt] = pltpu.async_copy(
                x_hbm.at[pl.ds(elem_base + (c + 1) * CHUNK_ELEMS, CHUNK_ELEMS)],
                xbufs[nxt],
                sems[nxt],
            )
        copies[cur].wait()
        xb = xbufs[cur]
        for g in range(GROUPS_PER_CHUNK):
            base_idx = (g * LANES + lane) * SEQ_N

            def s_step(i, acc, base_idx=base_idx, xb=xb):
                bi = base_idx + i * S_UNROLL
                for k in range(S_UNROLL):
                    tok = plsc.load_gather(xb, [bi + k])
                    pk = plsc.load_gather(twbuf, [lax.shift_right_logical(tok, 1)])
                    lo = lax.shift_left(pk, 16)
                    hi = lax.bitwise_and(pk, hi_mask)
                    bits = jnp.where(lax.bitwise_and(tok, 1) == 0, lo, hi)
                    acc = acc + plsc.bitcast(bits, jnp.float32)
                return acc

            acc = lax.fori_loop(
                0, SEQ_N // S_UNROLL, s_step, jnp.zeros((LANES,), jnp.float32)
            )
            z = acc * jnp.float32(1.0 / SEQ_N)
            res = 1.0 / (1.0 + jnp.exp(-z))
            outbuf[pl.ds(c * CHUNK_ROWS + g * LANES, LANES)] = res

    pltpu.sync_copy(outbuf, out_hbm.at[pl.ds(row_base, ROWS_PER_WORKER)])


@jax.jit
def kernel(x, table, W, b):
    tw = _compute_tw(table, W, b)
    packed = lax.bitcast_convert_type(
        tw.astype(jnp.bfloat16).reshape(NUM_PAIRS, 2), jnp.int32
    )
    x1d = x.reshape(BATCH_N * SEQ_N).astype(jnp.int32)

    mesh = plsc.VectorSubcoreMesh(
        core_axis_name="c",
        subcore_axis_name="s",
        num_cores=NUM_CORES,
        num_subcores=NUM_SUBCORES,
    )
    out1d = pl.kernel(
        _sc_body,
        out_type=jax.ShapeDtypeStruct((BATCH_N,), jnp.float32),
        mesh=mesh,
        compiler_params=pltpu.CompilerParams(needs_layout_passes=False),
        scratch_types=[
            pltpu.VMEM((NUM_PAIRS,), jnp.int32),
            pltpu.VMEM((CHUNK_ELEMS,), jnp.int32),
            pltpu.VMEM((CHUNK_ELEMS,), jnp.int32),
            pltpu.VMEM((ROWS_PER_WORKER,), jnp.float32),
            pltpu.SemaphoreType.DMA,
            pltpu.SemaphoreType.DMA,
            pltpu.SemaphoreType.DMA,
        ],
    )(packed, x1d)
    return out1d.reshape(BATCH_N, 1)
